# baseline (speedup 1.0000x reference)
import jax
import jax.numpy as jnp
from jax.experimental import pallas as pl

_PREC = jax.lax.Precision.HIGHEST


def _dot(a, b):
    return jnp.dot(a, b, precision=_PREC)

_N = 4096
_E = 8192
_DN = 11
_DE = 4
_G = 256


def _leaky(v):
    return jnp.where(v >= 0, v, 0.01 * v)


def _cnt_kernel(dst_r):
    EB = 512
    nE = _E // EB

    def body(dst_ref, out_ref):
        e = pl.program_id(0)
        dst = dst_ref[0, 0, :]
        oht = (jax.lax.broadcasted_iota(jnp.int32, (_N, EB), 0)
               == dst[None, :]).astype(jnp.float32)

        @pl.when(e == 0)
        def _():
            out_ref[...] = jnp.zeros_like(out_ref)

        out_ref[...] += _dot(oht, jnp.ones((EB, 8), jnp.float32))

    return pl.pallas_call(
        body,
        grid=(nE,),
        in_specs=[pl.BlockSpec((1, 1, EB), lambda e: (e, 0, 0))],
        out_specs=pl.BlockSpec((_N, 8), lambda e: (0, 0)),
        out_shape=jax.ShapeDtypeStruct((_N, 8), jnp.float32),
    )(dst_r)


def _gather(feats, src_r, EB=512):
    nE = _E // EB
    d = feats.shape[1]

    def body(src_ref, x_ref, out_ref):
        src = src_ref[0, 0, :]
        oh = (src[:, None] == jax.lax.broadcasted_iota(
            jnp.int32, (EB, _N), 1)).astype(jnp.float32)
        out_ref[...] = _dot(oh, x_ref[...])

    return pl.pallas_call(
        body,
        grid=(nE,),
        in_specs=[
            pl.BlockSpec((1, 1, EB), lambda e: (e, 0, 0)),
            pl.BlockSpec((_N, d), lambda e: (0, 0)),
        ],
        out_specs=pl.BlockSpec((EB, d), lambda e: (e, 0)),
        out_shape=jax.ShapeDtypeStruct((_E, d), jnp.float32),
    )(src_r, feats)


def _msg(dst_r, ea, xs, w1, b1, w2p, b2p, EB):
    hid = w1.shape[1]
    in_ch = xs.shape[1]
    nE = _E // EB
    OT = 128

    def body(dst_ref, ea_ref, xs_ref, w1_ref, b1_ref, w2_ref, b2_ref, out_ref):
        e = pl.program_id(0)
        h = _leaky(_dot(ea_ref[...], w1_ref[...]) + b1_ref[...])
        w2f = w2_ref[...].reshape(hid, in_ch * OT)
        we = _dot(h, w2f).reshape(EB, in_ch, OT) + b2_ref[...][None, :, :]
        msg = jnp.sum(xs_ref[...][:, :, None] * we, axis=1)
        dst = dst_ref[0, 0, :]
        oht = (jax.lax.broadcasted_iota(jnp.int32, (_N, EB), 0)
               == dst[None, :]).astype(jnp.float32)

        @pl.when(e == 0)
        def _():
            out_ref[...] = jnp.zeros_like(out_ref)

        out_ref[...] += _dot(oht, msg)

    return pl.pallas_call(
        body,
        grid=(nE,),
        in_specs=[
            pl.BlockSpec((1, 1, EB), lambda e: (e, 0, 0)),
            pl.BlockSpec((EB, _DE), lambda e: (e, 0)),
            pl.BlockSpec((EB, in_ch), lambda e: (e, 0)),
            pl.BlockSpec((_DE, hid), lambda e: (0, 0)),
            pl.BlockSpec((1, hid), lambda e: (0, 0)),
            pl.BlockSpec((hid, in_ch, OT), lambda e: (0, 0, 0)),
            pl.BlockSpec((in_ch, OT), lambda e: (0, 0)),
        ],
        out_specs=pl.BlockSpec((_N, OT), lambda e: (0, 0)),
        out_shape=jax.ShapeDtypeStruct((_N, OT), jnp.float32),
    )(dst_r, ea, xs, w1, b1, w2p, b2p)


def _combine(sums, cnt8, xin, root, bias):
    out_ch = sums.shape[1]

    def body(s_ref, c_ref, x_ref, r_ref, b_ref, out_ref):
        c = jnp.maximum(c_ref[...][:, :1], 1.0)
        out_ref[...] = _leaky(s_ref[...] / c + _dot(x_ref[...], r_ref[...]) + b_ref[...])

    return pl.pallas_call(
        body,
        out_shape=jax.ShapeDtypeStruct((_N, out_ch), jnp.float32),
    )(sums, cnt8, xin, root, bias[None, :])


def _head(batch_r, d3, params):
    w1 = params["fc1"]["w"]
    b1 = params["fc1"]["b"][None, :]
    w2 = params["fc2"]["w"]
    b2 = params["fc2"]["b"][None, :]
    w3p = jnp.pad(params["fc3"]["w"], ((0, 0), (0, 7)))
    b3p = jnp.pad(params["fc3"]["b"], (0, 7))[None, :]

    def body(b_ref, d_ref, w1_ref, b1_ref, w2_ref, b2_ref, w3_ref, b3_ref,
             out_ref):
        bidx = b_ref[0, 0, :]
        ohg = (jax.lax.broadcasted_iota(jnp.int32, (_G, _N), 0)
               == bidx[None, :]).astype(jnp.float32)
        sums = _dot(ohg, d_ref[...])
        cnt = jnp.maximum(jnp.sum(ohg, axis=1, keepdims=True), 1.0)
        pooled = sums / cnt
        h1 = _leaky(_dot(pooled, w1_ref[...]) + b1_ref[...])
        h2 = _leaky(_dot(h1, w2_ref[...]) + b2_ref[...])
        out_ref[...] = _dot(h2, w3_ref[...]) + b3_ref[...]

    out8 = pl.pallas_call(
        body,
        out_shape=jax.ShapeDtypeStruct((_G, 8), jnp.float32),
    )(batch_r, d3, w1, b1, w2, b2, w3p, b3p)
    return out8[:, :1]


def kernel(x, edge_index, edge_attr, batch, params):
    src = edge_index[0]
    dst = edge_index[1]
    EBs = 512
    src_r = src.reshape(_E // EBs, 1, EBs)
    dst_r = dst.reshape(_E // EBs, 1, EBs)
    batch_r = batch.reshape(1, 1, _N)

    cnt8 = _cnt_kernel(dst_r)

    feats = x
    for name, in_ch, out_ch, EBm in (
            ("conv1", _DN, 128, 512),
            ("conv2", 128 + _DN, 128, 128),
            ("conv3", 128 + _DN, 256, 128),
    ):
        p = params[name]
        hid = p["w1"].shape[1]
        w2p = p["w2"].reshape(hid, in_ch, out_ch)
        b2p = p["b2"].reshape(in_ch, out_ch)
        xs = _gather(feats, src_r)
        dst_rm = dst.reshape(_E // EBm, 1, EBm)
        halves = []
        for ot in range(out_ch // 128):
            halves.append(_msg(
                dst_rm, edge_attr, xs,
                p["w1"], p["b1"][None, :],
                w2p[:, :, ot * 128:(ot + 1) * 128],
                b2p[:, ot * 128:(ot + 1) * 128],
                EBm,
            ))
        sums = halves[0] if len(halves) == 1 else jnp.concatenate(halves, axis=1)
        d = _combine(sums, cnt8, feats, p["root"], p["bias"])
        feats = jnp.concatenate([d, x], axis=1)

    return _head(batch_r, feats, params)


# speedup vs baseline: 1.9831x; 1.9831x over previous
import jax
import jax.numpy as jnp
from jax.experimental import pallas as pl

_PREC = jax.lax.Precision.HIGHEST


def _dot(a, b):
    return jnp.dot(a, b, precision=_PREC)


def _split_bf16(a):
    hi = a.astype(jnp.bfloat16)
    lo = (a - hi.astype(jnp.float32)).astype(jnp.bfloat16)
    return hi, lo


def _dot_exact_lhs(lhs_bf, rhs_f32):
    hi, lo = _split_bf16(rhs_f32)
    return (jnp.dot(lhs_bf, hi, preferred_element_type=jnp.float32)
            + jnp.dot(lhs_bf, lo, preferred_element_type=jnp.float32))

_N = 4096
_E = 8192
_DN = 11
_DE = 4
_G = 256


def _leaky(v):
    return jnp.where(v >= 0, v, 0.01 * v)


def _cnt_kernel(dst_r):
    EB = 512
    nE = _E // EB

    def body(dst_ref, out_ref):
        e = pl.program_id(0)
        dst = dst_ref[0, 0, :]
        oht = (jax.lax.broadcasted_iota(jnp.int32, (_N, EB), 0)
               == dst[None, :]).astype(jnp.bfloat16)

        @pl.when(e == 0)
        def _():
            out_ref[...] = jnp.zeros_like(out_ref)

        out_ref[...] += jnp.dot(oht, jnp.ones((EB, 8), jnp.bfloat16),
                                preferred_element_type=jnp.float32)

    return pl.pallas_call(
        body,
        grid=(nE,),
        in_specs=[pl.BlockSpec((1, 1, EB), lambda e: (e, 0, 0))],
        out_specs=pl.BlockSpec((_N, 8), lambda e: (0, 0)),
        out_shape=jax.ShapeDtypeStruct((_N, 8), jnp.float32),
    )(dst_r)


def _gather(feats, src_r, EB=512):
    nE = _E // EB
    d = feats.shape[1]

    def body(src_ref, x_ref, out_ref):
        src = src_ref[0, 0, :]
        oh = (src[:, None] == jax.lax.broadcasted_iota(
            jnp.int32, (EB, _N), 1)).astype(jnp.bfloat16)
        out_ref[...] = _dot_exact_lhs(oh, x_ref[...])

    return pl.pallas_call(
        body,
        grid=(nE,),
        in_specs=[
            pl.BlockSpec((1, 1, EB), lambda e: (e, 0, 0)),
            pl.BlockSpec((_N, d), lambda e: (0, 0)),
        ],
        out_specs=pl.BlockSpec((EB, d), lambda e: (e, 0)),
        out_shape=jax.ShapeDtypeStruct((_E, d), jnp.float32),
    )(src_r, feats)


def _msg(dst_r, ea, xs, w1, b1, w2p, b2p, EB):
    hid = w1.shape[1]
    in_ch = xs.shape[1]
    nE = _E // EB
    OT = 128

    def body(dst_ref, ea_ref, xs_ref, w1_ref, b1_ref, w2_ref, b2_ref, out_ref):
        e = pl.program_id(0)
        h = _leaky(_dot(ea_ref[...], w1_ref[...]) + b1_ref[...])
        h_hi, h_lo = _split_bf16(h)
        w2f = w2_ref[...].reshape(hid, in_ch * OT)
        we = (jnp.dot(h_hi, w2f, preferred_element_type=jnp.float32)
              + jnp.dot(h_lo, w2f, preferred_element_type=jnp.float32)
              ).reshape(EB, in_ch, OT) + b2_ref[...][None, :, :]
        msg = jnp.sum(xs_ref[...][:, :, None] * we, axis=1)
        dst = dst_ref[0, 0, :]
        oht = (jax.lax.broadcasted_iota(jnp.int32, (_N, EB), 0)
               == dst[None, :]).astype(jnp.bfloat16)

        @pl.when(e == 0)
        def _():
            out_ref[...] = jnp.zeros_like(out_ref)

        out_ref[...] += _dot_exact_lhs(oht, msg)

    return pl.pallas_call(
        body,
        grid=(nE,),
        in_specs=[
            pl.BlockSpec((1, 1, EB), lambda e: (e, 0, 0)),
            pl.BlockSpec((EB, _DE), lambda e: (e, 0)),
            pl.BlockSpec((EB, in_ch), lambda e: (e, 0)),
            pl.BlockSpec((_DE, hid), lambda e: (0, 0)),
            pl.BlockSpec((1, hid), lambda e: (0, 0)),
            pl.BlockSpec((hid, in_ch, OT), lambda e: (0, 0, 0)),
            pl.BlockSpec((in_ch, OT), lambda e: (0, 0)),
        ],
        out_specs=pl.BlockSpec((_N, OT), lambda e: (0, 0)),
        out_shape=jax.ShapeDtypeStruct((_N, OT), jnp.float32),
    )(dst_r, ea, xs, w1, b1, w2p, b2p)


def _combine(sums, cnt8, xin, root, bias):
    out_ch = sums.shape[1]

    def body(s_ref, c_ref, x_ref, r_ref, b_ref, out_ref):
        c = jnp.maximum(c_ref[...][:, :1], 1.0)
        out_ref[...] = _leaky(s_ref[...] / c + _dot(x_ref[...], r_ref[...]) + b_ref[...])

    return pl.pallas_call(
        body,
        out_shape=jax.ShapeDtypeStruct((_N, out_ch), jnp.float32),
    )(sums, cnt8, xin, root, bias[None, :])


def _head(batch_r, d3, params):
    w1 = params["fc1"]["w"]
    b1 = params["fc1"]["b"][None, :]
    w2 = params["fc2"]["w"]
    b2 = params["fc2"]["b"][None, :]
    w3p = jnp.pad(params["fc3"]["w"], ((0, 0), (0, 7)))
    b3p = jnp.pad(params["fc3"]["b"], (0, 7))[None, :]

    def body(b_ref, d_ref, w1_ref, b1_ref, w2_ref, b2_ref, w3_ref, b3_ref,
             out_ref):
        bidx = b_ref[0, 0, :]
        ohg = (jax.lax.broadcasted_iota(jnp.int32, (_G, _N), 0)
               == bidx[None, :]).astype(jnp.float32)
        sums = _dot(ohg, d_ref[...])
        cnt = jnp.maximum(jnp.sum(ohg, axis=1, keepdims=True), 1.0)
        pooled = sums / cnt
        h1 = _leaky(_dot(pooled, w1_ref[...]) + b1_ref[...])
        h2 = _leaky(_dot(h1, w2_ref[...]) + b2_ref[...])
        out_ref[...] = _dot(h2, w3_ref[...]) + b3_ref[...]

    out8 = pl.pallas_call(
        body,
        out_shape=jax.ShapeDtypeStruct((_G, 8), jnp.float32),
    )(batch_r, d3, w1, b1, w2, b2, w3p, b3p)
    return out8[:, :1]


def kernel(x, edge_index, edge_attr, batch, params):
    src = edge_index[0]
    dst = edge_index[1]
    EBs = 512
    src_r = src.reshape(_E // EBs, 1, EBs)
    dst_r = dst.reshape(_E // EBs, 1, EBs)
    batch_r = batch.reshape(1, 1, _N)

    cnt8 = _cnt_kernel(dst_r)

    feats = x
    for name, in_ch, out_ch, EBm in (
            ("conv1", _DN, 128, 512),
            ("conv2", 128 + _DN, 128, 128),
            ("conv3", 128 + _DN, 256, 128),
    ):
        p = params[name]
        hid = p["w1"].shape[1]
        w2p = p["w2"].reshape(hid, in_ch, out_ch).astype(jnp.bfloat16)
        b2p = p["b2"].reshape(in_ch, out_ch)
        xs = _gather(feats, src_r)
        dst_rm = dst.reshape(_E // EBm, 1, EBm)
        halves = []
        for ot in range(out_ch // 128):
            halves.append(_msg(
                dst_rm, edge_attr, xs,
                p["w1"], p["b1"][None, :],
                w2p[:, :, ot * 128:(ot + 1) * 128],
                b2p[:, ot * 128:(ot + 1) * 128],
                EBm,
            ))
        sums = halves[0] if len(halves) == 1 else jnp.concatenate(halves, axis=1)
        d = _combine(sums, cnt8, feats, p["root"], p["bias"])
        feats = jnp.concatenate([d, x], axis=1)

    return _head(batch_r, feats, params)


# speedup vs baseline: 2.1183x; 1.0682x over previous
import jax
import jax.numpy as jnp
from jax.experimental import pallas as pl

_PREC = jax.lax.Precision.HIGHEST


def _dot(a, b):
    return jnp.dot(a, b, precision=_PREC)


def _split_bf16(a):
    hi = a.astype(jnp.bfloat16)
    lo = (a - hi.astype(jnp.float32)).astype(jnp.bfloat16)
    return hi, lo


def _dot_exact_lhs(lhs_bf, rhs_f32):
    hi, lo = _split_bf16(rhs_f32)
    return (jnp.dot(lhs_bf, hi, preferred_element_type=jnp.float32)
            + jnp.dot(lhs_bf, lo, preferred_element_type=jnp.float32))

_N = 4096
_E = 8192
_DN = 11
_DE = 4
_G = 256


def _leaky(v):
    return jnp.where(v >= 0, v, 0.01 * v)


def _cnt_kernel(dst_r):
    EB = 512
    nE = _E // EB

    def body(dst_ref, out_ref):
        e = pl.program_id(0)
        dst = dst_ref[0, 0, :]
        oht = (jax.lax.broadcasted_iota(jnp.int32, (_N, EB), 0)
               == dst[None, :]).astype(jnp.bfloat16)

        @pl.when(e == 0)
        def _():
            out_ref[...] = jnp.zeros_like(out_ref)

        out_ref[...] += jnp.dot(oht, jnp.ones((EB, 8), jnp.bfloat16),
                                preferred_element_type=jnp.float32)

    return pl.pallas_call(
        body,
        grid=(nE,),
        in_specs=[pl.BlockSpec((1, 1, EB), lambda e: (e, 0, 0))],
        out_specs=pl.BlockSpec((_N, 8), lambda e: (0, 0)),
        out_shape=jax.ShapeDtypeStruct((_N, 8), jnp.float32),
    )(dst_r)


def _gather(feats, src_r, EB=512):
    nE = _E // EB
    d = feats.shape[1]

    def body(src_ref, x_ref, out_ref):
        src = src_ref[0, 0, :]
        oh = (src[:, None] == jax.lax.broadcasted_iota(
            jnp.int32, (EB, _N), 1)).astype(jnp.bfloat16)
        out_ref[...] = _dot_exact_lhs(oh, x_ref[...])

    return pl.pallas_call(
        body,
        grid=(nE,),
        in_specs=[
            pl.BlockSpec((1, 1, EB), lambda e: (e, 0, 0)),
            pl.BlockSpec((_N, d), lambda e: (0, 0)),
        ],
        out_specs=pl.BlockSpec((EB, d), lambda e: (e, 0)),
        out_shape=jax.ShapeDtypeStruct((_E, d), jnp.float32),
    )(src_r, feats)


def _msg(dst_r, ea, xs, w1, b1, w2p, b2p, EB):
    hid = w1.shape[1]
    in_ch = xs.shape[1]
    nE = _E // EB
    OT = 128

    def body(dst_ref, ea_ref, xs_ref, w1_ref, b1_ref, w2_ref, b2_ref, out_ref):
        e = pl.program_id(0)
        h = _leaky(_dot(ea_ref[...], w1_ref[...]) + b1_ref[...])
        w2f = w2_ref[...].reshape(hid, in_ch * OT)
        we = jnp.dot(h.astype(jnp.bfloat16), w2f,
                     preferred_element_type=jnp.float32
                     ).reshape(EB, in_ch, OT) + b2_ref[...][None, :, :]
        msg = jnp.sum(xs_ref[...][:, :, None] * we, axis=1)
        dst = dst_ref[0, 0, :]
        oht = (jax.lax.broadcasted_iota(jnp.int32, (_N, EB), 0)
               == dst[None, :]).astype(jnp.bfloat16)

        @pl.when(e == 0)
        def _():
            out_ref[...] = jnp.zeros_like(out_ref)

        out_ref[...] += _dot_exact_lhs(oht, msg)

    return pl.pallas_call(
        body,
        grid=(nE,),
        in_specs=[
            pl.BlockSpec((1, 1, EB), lambda e: (e, 0, 0)),
            pl.BlockSpec((EB, _DE), lambda e: (e, 0)),
            pl.BlockSpec((EB, in_ch), lambda e: (e, 0)),
            pl.BlockSpec((_DE, hid), lambda e: (0, 0)),
            pl.BlockSpec((1, hid), lambda e: (0, 0)),
            pl.BlockSpec((hid, in_ch, OT), lambda e: (0, 0, 0)),
            pl.BlockSpec((in_ch, OT), lambda e: (0, 0)),
        ],
        out_specs=pl.BlockSpec((_N, OT), lambda e: (0, 0)),
        out_shape=jax.ShapeDtypeStruct((_N, OT), jnp.float32),
    )(dst_r, ea, xs, w1, b1, w2p, b2p)


def _combine(sums, cnt8, xin, root, bias):
    out_ch = sums.shape[1]

    def body(s_ref, c_ref, x_ref, r_ref, b_ref, out_ref):
        c = jnp.maximum(c_ref[...][:, :1], 1.0)
        out_ref[...] = _leaky(s_ref[...] / c + _dot(x_ref[...], r_ref[...]) + b_ref[...])

    return pl.pallas_call(
        body,
        out_shape=jax.ShapeDtypeStruct((_N, out_ch), jnp.float32),
    )(sums, cnt8, xin, root, bias[None, :])


def _head(batch_r, d3, params):
    w1 = params["fc1"]["w"]
    b1 = params["fc1"]["b"][None, :]
    w2 = params["fc2"]["w"]
    b2 = params["fc2"]["b"][None, :]
    w3p = jnp.pad(params["fc3"]["w"], ((0, 0), (0, 7)))
    b3p = jnp.pad(params["fc3"]["b"], (0, 7))[None, :]

    def body(b_ref, d_ref, w1_ref, b1_ref, w2_ref, b2_ref, w3_ref, b3_ref,
             out_ref):
        bidx = b_ref[0, 0, :]
        ohg = (jax.lax.broadcasted_iota(jnp.int32, (_G, _N), 0)
               == bidx[None, :]).astype(jnp.float32)
        sums = _dot(ohg, d_ref[...])
        cnt = jnp.maximum(jnp.sum(ohg, axis=1, keepdims=True), 1.0)
        pooled = sums / cnt
        h1 = _leaky(_dot(pooled, w1_ref[...]) + b1_ref[...])
        h2 = _leaky(_dot(h1, w2_ref[...]) + b2_ref[...])
        out_ref[...] = _dot(h2, w3_ref[...]) + b3_ref[...]

    out8 = pl.pallas_call(
        body,
        out_shape=jax.ShapeDtypeStruct((_G, 8), jnp.float32),
    )(batch_r, d3, w1, b1, w2, b2, w3p, b3p)
    return out8[:, :1]


def kernel(x, edge_index, edge_attr, batch, params):
    src = edge_index[0]
    dst = edge_index[1]
    EBs = 512
    src_r = src.reshape(_E // EBs, 1, EBs)
    dst_r = dst.reshape(_E // EBs, 1, EBs)
    batch_r = batch.reshape(1, 1, _N)

    cnt8 = _cnt_kernel(dst_r)

    feats = x
    for name, in_ch, out_ch, EBm in (
            ("conv1", _DN, 128, 512),
            ("conv2", 128 + _DN, 128, 128),
            ("conv3", 128 + _DN, 256, 128),
    ):
        p = params[name]
        hid = p["w1"].shape[1]
        w2p = p["w2"].reshape(hid, in_ch, out_ch).astype(jnp.bfloat16)
        b2p = p["b2"].reshape(in_ch, out_ch)
        xs = _gather(feats, src_r)
        dst_rm = dst.reshape(_E // EBm, 1, EBm)
        halves = []
        for ot in range(out_ch // 128):
            halves.append(_msg(
                dst_rm, edge_attr, xs,
                p["w1"], p["b1"][None, :],
                w2p[:, :, ot * 128:(ot + 1) * 128],
                b2p[:, ot * 128:(ot + 1) * 128],
                EBm,
            ))
        sums = halves[0] if len(halves) == 1 else jnp.concatenate(halves, axis=1)
        d = _combine(sums, cnt8, feats, p["root"], p["bias"])
        feats = jnp.concatenate([d, x], axis=1)

    return _head(batch_r, feats, params)


# speedup vs baseline: 2.5867x; 1.2211x over previous
import jax
import jax.numpy as jnp
from jax.experimental import pallas as pl

_PREC = jax.lax.Precision.HIGHEST


def _dot(a, b):
    return jnp.dot(a, b, precision=_PREC)


def _split_bf16(a):
    hi = a.astype(jnp.bfloat16)
    lo = (a - hi.astype(jnp.float32)).astype(jnp.bfloat16)
    return hi, lo


def _dot_exact_lhs(lhs_bf, rhs_f32):
    hi, lo = _split_bf16(rhs_f32)
    return (jnp.dot(lhs_bf, hi, preferred_element_type=jnp.float32)
            + jnp.dot(lhs_bf, lo, preferred_element_type=jnp.float32))

_N = 4096
_E = 8192
_DN = 11
_DE = 4
_G = 256


def _leaky(v):
    return jnp.where(v >= 0, v, 0.01 * v)


def _cnt_kernel(dst_r):
    EB = 512
    nE = _E // EB

    def body(dst_ref, out_ref):
        e = pl.program_id(0)
        dst = dst_ref[0, 0, :]
        oht = (jax.lax.broadcasted_iota(jnp.int32, (_N, EB), 0)
               == dst[None, :]).astype(jnp.bfloat16)

        @pl.when(e == 0)
        def _():
            out_ref[...] = jnp.zeros_like(out_ref)

        out_ref[...] += jnp.dot(oht, jnp.ones((EB, 8), jnp.bfloat16),
                                preferred_element_type=jnp.float32)

    return pl.pallas_call(
        body,
        grid=(nE,),
        in_specs=[pl.BlockSpec((1, 1, EB), lambda e: (e, 0, 0))],
        out_specs=pl.BlockSpec((_N, 8), lambda e: (0, 0)),
        out_shape=jax.ShapeDtypeStruct((_N, 8), jnp.float32),
    )(dst_r)


def _gather(feats, src_r, EB=512):
    nE = _E // EB
    d = feats.shape[1]

    def body(src_ref, x_ref, out_ref):
        src = src_ref[0, 0, :]
        oh = (src[:, None] == jax.lax.broadcasted_iota(
            jnp.int32, (EB, _N), 1)).astype(jnp.bfloat16)
        out_ref[...] = _dot_exact_lhs(oh, x_ref[...])

    return pl.pallas_call(
        body,
        grid=(nE,),
        in_specs=[
            pl.BlockSpec((1, 1, EB), lambda e: (e, 0, 0)),
            pl.BlockSpec((_N, d), lambda e: (0, 0)),
        ],
        out_specs=pl.BlockSpec((EB, d), lambda e: (e, 0)),
        out_shape=jax.ShapeDtypeStruct((_E, d), jnp.float32),
    )(src_r, feats)


def _msg(dst_r, ea, xs, w1, b1, w2p, b2p, EB):
    hid = w1.shape[1]
    in_ch = xs.shape[1]
    nE = _E // EB
    OT = 128

    def body(dst_ref, ea_ref, xs_ref, w1_ref, b1_ref, w2_ref, b2_ref, out_ref):
        e = pl.program_id(0)
        h = _leaky(_dot(ea_ref[...], w1_ref[...]) + b1_ref[...])
        w2f = w2_ref[...].reshape(hid, in_ch * OT)
        we = jnp.dot(h.astype(jnp.bfloat16), w2f,
                     preferred_element_type=jnp.float32
                     ).reshape(EB, in_ch, OT) + b2_ref[...][None, :, :]
        msg = jnp.sum(xs_ref[...][:, :, None] * we, axis=1)
        dst = dst_ref[0, 0, :]
        oht = (jax.lax.broadcasted_iota(jnp.int32, (_N, EB), 0)
               == dst[None, :]).astype(jnp.bfloat16)

        @pl.when(e == 0)
        def _():
            out_ref[...] = jnp.zeros_like(out_ref)

        out_ref[...] += _dot_exact_lhs(oht, msg)

    return pl.pallas_call(
        body,
        grid=(nE,),
        in_specs=[
            pl.BlockSpec((1, 1, EB), lambda e: (e, 0, 0)),
            pl.BlockSpec((EB, _DE), lambda e: (e, 0)),
            pl.BlockSpec((EB, in_ch), lambda e: (e, 0)),
            pl.BlockSpec((_DE, hid), lambda e: (0, 0)),
            pl.BlockSpec((1, hid), lambda e: (0, 0)),
            pl.BlockSpec((hid, in_ch, OT), lambda e: (0, 0, 0)),
            pl.BlockSpec((in_ch, OT), lambda e: (0, 0)),
        ],
        out_specs=pl.BlockSpec((_N, OT), lambda e: (0, 0)),
        out_shape=jax.ShapeDtypeStruct((_N, OT), jnp.float32),
    )(dst_r, ea, xs, w1, b1, w2p, b2p)


def _combine(sums, cnt8, xin, root, bias):
    out_ch = sums.shape[1]

    def body(s_ref, c_ref, x_ref, r_ref, b_ref, out_ref):
        c = jnp.maximum(c_ref[...][:, :1], 1.0)
        out_ref[...] = _leaky(s_ref[...] / c + _dot(x_ref[...], r_ref[...]) + b_ref[...])

    return pl.pallas_call(
        body,
        out_shape=jax.ShapeDtypeStruct((_N, out_ch), jnp.float32),
    )(sums, cnt8, xin, root, bias[None, :])


def _head(batch_r, d3, params):
    w1 = params["fc1"]["w"]
    b1 = params["fc1"]["b"][None, :]
    w2 = params["fc2"]["w"]
    b2 = params["fc2"]["b"][None, :]
    w3p = jnp.pad(params["fc3"]["w"], ((0, 0), (0, 7)))
    b3p = jnp.pad(params["fc3"]["b"], (0, 7))[None, :]

    def body(b_ref, d_ref, w1_ref, b1_ref, w2_ref, b2_ref, w3_ref, b3_ref,
             out_ref):
        bidx = b_ref[0, 0, :]
        ohg = (jax.lax.broadcasted_iota(jnp.int32, (_G, _N), 0)
               == bidx[None, :]).astype(jnp.float32)
        sums = _dot(ohg, d_ref[...])
        cnt = jnp.maximum(jnp.sum(ohg, axis=1, keepdims=True), 1.0)
        pooled = sums / cnt
        h1 = _leaky(_dot(pooled, w1_ref[...]) + b1_ref[...])
        h2 = _leaky(_dot(h1, w2_ref[...]) + b2_ref[...])
        out_ref[...] = _dot(h2, w3_ref[...]) + b3_ref[...]

    out8 = pl.pallas_call(
        body,
        out_shape=jax.ShapeDtypeStruct((_G, 8), jnp.float32),
    )(batch_r, d3, w1, b1, w2, b2, w3p, b3p)
    return out8[:, :1]


def kernel(x, edge_index, edge_attr, batch, params):
    src = edge_index[0]
    dst = edge_index[1]
    EBs = 512
    src_r = src.reshape(_E // EBs, 1, EBs)
    dst_r = dst.reshape(_E // EBs, 1, EBs)
    batch_r = batch.reshape(1, 1, _N)

    cnt8 = _cnt_kernel(dst_r)

    feats = x
    for name, in_ch, out_ch, EBm in (
            ("conv1", _DN, 128, 512),
            ("conv2", 128 + _DN, 128, 256),
            ("conv3", 128 + _DN, 256, 256),
    ):
        p = params[name]
        hid = p["w1"].shape[1]
        w2p = p["w2"].reshape(hid, in_ch, out_ch).astype(jnp.bfloat16)
        b2p = p["b2"].reshape(in_ch, out_ch)
        xs = _gather(feats, src_r)
        dst_rm = dst.reshape(_E // EBm, 1, EBm)
        halves = []
        for ot in range(out_ch // 128):
            halves.append(_msg(
                dst_rm, edge_attr, xs,
                p["w1"], p["b1"][None, :],
                w2p[:, :, ot * 128:(ot + 1) * 128],
                b2p[:, ot * 128:(ot + 1) * 128],
                EBm,
            ))
        sums = halves[0] if len(halves) == 1 else jnp.concatenate(halves, axis=1)
        d = _combine(sums, cnt8, feats, p["root"], p["bias"])
        feats = jnp.concatenate([d, x], axis=1)

    return _head(batch_r, feats, params)


# speedup vs baseline: 2.9942x; 1.1576x over previous
import jax
import jax.numpy as jnp
from jax.experimental import pallas as pl

_PREC = jax.lax.Precision.HIGHEST


def _dot(a, b):
    return jnp.dot(a, b, precision=_PREC)


def _split_bf16(a):
    hi = a.astype(jnp.bfloat16)
    lo = (a - hi.astype(jnp.float32)).astype(jnp.bfloat16)
    return hi, lo


def _dot_exact_lhs(lhs_bf, rhs_f32):
    hi, lo = _split_bf16(rhs_f32)
    return (jnp.dot(lhs_bf, hi, preferred_element_type=jnp.float32)
            + jnp.dot(lhs_bf, lo, preferred_element_type=jnp.float32))

_N = 4096
_E = 8192
_DN = 11
_DE = 4
_G = 256


def _leaky(v):
    return jnp.where(v >= 0, v, 0.01 * v)


def _cnt_kernel(dst_r):
    EB = 512
    nE = _E // EB

    def body(dst_ref, out_ref):
        e = pl.program_id(0)
        dst = dst_ref[0, 0, :]
        oht = (jax.lax.broadcasted_iota(jnp.int32, (_N, EB), 0)
               == dst[None, :]).astype(jnp.bfloat16)

        @pl.when(e == 0)
        def _():
            out_ref[...] = jnp.zeros_like(out_ref)

        out_ref[...] += jnp.dot(oht, jnp.ones((EB, 8), jnp.bfloat16),
                                preferred_element_type=jnp.float32)

    return pl.pallas_call(
        body,
        grid=(nE,),
        in_specs=[pl.BlockSpec((1, 1, EB), lambda e: (e, 0, 0))],
        out_specs=pl.BlockSpec((_N, 8), lambda e: (0, 0)),
        out_shape=jax.ShapeDtypeStruct((_N, 8), jnp.float32),
    )(dst_r)


def _gather(feats, src_r, EB=512):
    nE = _E // EB
    d = feats.shape[1]

    def body(src_ref, x_ref, out_ref):
        src = src_ref[0, 0, :]
        oh = (src[:, None] == jax.lax.broadcasted_iota(
            jnp.int32, (EB, _N), 1)).astype(jnp.bfloat16)
        out_ref[...] = _dot_exact_lhs(oh, x_ref[...])

    return pl.pallas_call(
        body,
        grid=(nE,),
        in_specs=[
            pl.BlockSpec((1, 1, EB), lambda e: (e, 0, 0)),
            pl.BlockSpec((_N, d), lambda e: (0, 0)),
        ],
        out_specs=pl.BlockSpec((EB, d), lambda e: (e, 0)),
        out_shape=jax.ShapeDtypeStruct((_E, d), jnp.float32),
    )(src_r, feats)


def _msg(dst_r, ea, xs, w1, b1, w2p, b2p, EB):
    hid = w1.shape[1]
    in_ch = xs.shape[1]
    nE = _E // EB
    OT = 128

    def body(dst_ref, ea_ref, xs_ref, w1_ref, b1_ref, w2_ref, b2_ref, out_ref):
        e = pl.program_id(0)
        h = _leaky(_dot(ea_ref[...], w1_ref[...]) + b1_ref[...])
        w2f = w2_ref[...].reshape(hid, in_ch * OT)
        we = jnp.dot(h.astype(jnp.bfloat16), w2f,
                     preferred_element_type=jnp.float32
                     ).reshape(EB, in_ch, OT) + b2_ref[...][None, :, :]
        msg = jnp.sum(xs_ref[...][:, :, None] * we, axis=1)
        dst = dst_ref[0, 0, :]
        oht = (jax.lax.broadcasted_iota(jnp.int32, (_N, EB), 0)
               == dst[None, :]).astype(jnp.bfloat16)

        @pl.when(e == 0)
        def _():
            out_ref[...] = jnp.zeros_like(out_ref)

        out_ref[...] += _dot_exact_lhs(oht, msg)

    return pl.pallas_call(
        body,
        grid=(nE,),
        in_specs=[
            pl.BlockSpec((1, 1, EB), lambda e: (e, 0, 0)),
            pl.BlockSpec((EB, _DE), lambda e: (e, 0)),
            pl.BlockSpec((EB, in_ch), lambda e: (e, 0)),
            pl.BlockSpec((_DE, hid), lambda e: (0, 0)),
            pl.BlockSpec((1, hid), lambda e: (0, 0)),
            pl.BlockSpec((hid, in_ch, OT), lambda e: (0, 0, 0)),
            pl.BlockSpec((in_ch, OT), lambda e: (0, 0)),
        ],
        out_specs=pl.BlockSpec((_N, OT), lambda e: (0, 0)),
        out_shape=jax.ShapeDtypeStruct((_N, OT), jnp.float32),
    )(dst_r, ea, xs, w1, b1, w2p, b2p)


def _combine(sums, cnt8, xin, root, bias):
    out_ch = sums.shape[1]

    def body(s_ref, c_ref, x_ref, r_ref, b_ref, out_ref):
        c = jnp.maximum(c_ref[...][:, :1], 1.0)
        out_ref[...] = _leaky(s_ref[...] / c + _dot(x_ref[...], r_ref[...]) + b_ref[...])

    return pl.pallas_call(
        body,
        out_shape=jax.ShapeDtypeStruct((_N, out_ch), jnp.float32),
    )(sums, cnt8, xin, root, bias[None, :])


def _head(batch_r, d3, params):
    w1 = params["fc1"]["w"]
    b1 = params["fc1"]["b"][None, :]
    w2 = params["fc2"]["w"]
    b2 = params["fc2"]["b"][None, :]
    w3p = jnp.pad(params["fc3"]["w"], ((0, 0), (0, 7)))
    b3p = jnp.pad(params["fc3"]["b"], (0, 7))[None, :]

    def body(b_ref, d_ref, w1_ref, b1_ref, w2_ref, b2_ref, w3_ref, b3_ref,
             out_ref):
        bidx = b_ref[0, 0, :]
        ohg = (jax.lax.broadcasted_iota(jnp.int32, (_G, _N), 0)
               == bidx[None, :]).astype(jnp.float32)
        sums = _dot(ohg, d_ref[...])
        cnt = jnp.maximum(jnp.sum(ohg, axis=1, keepdims=True), 1.0)
        pooled = sums / cnt
        h1 = _leaky(_dot(pooled, w1_ref[...]) + b1_ref[...])
        h2 = _leaky(_dot(h1, w2_ref[...]) + b2_ref[...])
        out_ref[...] = _dot(h2, w3_ref[...]) + b3_ref[...]

    out8 = pl.pallas_call(
        body,
        out_shape=jax.ShapeDtypeStruct((_G, 8), jnp.float32),
    )(batch_r, d3, w1, b1, w2, b2, w3p, b3p)
    return out8[:, :1]


def kernel(x, edge_index, edge_attr, batch, params):
    src = edge_index[0]
    dst = edge_index[1]
    EBs = 512
    src_r = src.reshape(_E // EBs, 1, EBs)
    dst_r = dst.reshape(_E // EBs, 1, EBs)
    batch_r = batch.reshape(1, 1, _N)

    cnt8 = _cnt_kernel(dst_r)

    feats = x
    for name, in_ch, out_ch, EBm in (
            ("conv1", _DN, 128, 512),
            ("conv2", 128 + _DN, 128, 512),
            ("conv3", 128 + _DN, 256, 512),
    ):
        p = params[name]
        hid = p["w1"].shape[1]
        w2p = p["w2"].reshape(hid, in_ch, out_ch).astype(jnp.bfloat16)
        b2p = p["b2"].reshape(in_ch, out_ch)
        xs = _gather(feats, src_r)
        dst_rm = dst.reshape(_E // EBm, 1, EBm)
        halves = []
        for ot in range(out_ch // 128):
            halves.append(_msg(
                dst_rm, edge_attr, xs,
                p["w1"], p["b1"][None, :],
                w2p[:, :, ot * 128:(ot + 1) * 128],
                b2p[:, ot * 128:(ot + 1) * 128],
                EBm,
            ))
        sums = halves[0] if len(halves) == 1 else jnp.concatenate(halves, axis=1)
        d = _combine(sums, cnt8, feats, p["root"], p["bias"])
        feats = jnp.concatenate([d, x], axis=1)

    return _head(batch_r, feats, params)
